# baseline (speedup 1.0000x reference)
import functools
import jax
import jax.numpy as jnp
from jax import lax
from jax.experimental import pallas as pl

K_SIZE = 2
DIM = 3
CHANNELS = [1, 16, 32, 32, 32, 64, 64, 64]
NUM_CLASSES = 10
GRID_B = 4


def _bn_elu_body(h_ref, m_ref, n_ref, g_ref, b_ref, o_ref):
    h = h_ref[...]
    m = m_ref[...]
    n = n_ref[0, 0]
    e = jnp.where(h > 0, h, jnp.exp(jnp.minimum(h, 0.0)) - 1.0)
    mu = jnp.sum(e * m, axis=0, keepdims=True) / n
    d = e - mu
    var = jnp.sum(jnp.where(m > 0, d * d, 0.0), axis=0, keepdims=True) / n
    o_ref[...] = d * jax.lax.rsqrt(var + 1e-5) * g_ref[...] + b_ref[...]


def _bn_elu_res_body(h_ref, m_ref, n_ref, g_ref, b_ref, r_ref, o_ref):
    h = h_ref[...]
    m = m_ref[...]
    n = n_ref[0, 0]
    e = jnp.where(h > 0, h, jnp.exp(jnp.minimum(h, 0.0)) - 1.0)
    mu = jnp.sum(e * m, axis=0, keepdims=True) / n
    d = e - mu
    var = jnp.sum(jnp.where(m > 0, d * d, 0.0), axis=0, keepdims=True) / n
    o_ref[...] = d * jax.lax.rsqrt(var + 1e-5) * g_ref[...] + b_ref[...] + r_ref[...]


def _bn_elu(h, mask, n, g, b, res=None):
    N, C = h.shape
    out_shape = jax.ShapeDtypeStruct((N, C), h.dtype)
    n2 = jnp.reshape(n, (1, 1))
    m2 = mask[:, None]
    g2 = g[None, :]
    b2 = b[None, :]
    if res is None:
        return pl.pallas_call(_bn_elu_body, out_shape=out_shape)(h, m2, n2, g2, b2)
    return pl.pallas_call(_bn_elu_res_body, out_shape=out_shape)(h, m2, n2, g2, b2, res)


def _coarsen(src, dst, valid, N):
    sentinel = N * N
    key = jnp.where(valid & (src != dst), src * N + dst, sentinel)
    key = jnp.sort(key)
    first = jnp.concatenate([jnp.ones((1,), dtype=bool), key[1:] != key[:-1]])
    valid2 = first & (key != sentinel)
    src2 = jnp.where(valid2, key // N, 0)
    dst2 = jnp.where(valid2, key % N, N)
    return src2, dst2, valid2


def _cart_norm(p, src, dst, valid):
    cart = jnp.where(valid[:, None], p[jnp.minimum(dst, p.shape[0] - 1)] - p[src], 0.0)
    m = jnp.max(jnp.abs(cart))
    return cart / (2.0 * m) + 0.5


def _sconv(x, src, dst, valid, pseudo, W, N):
    frac = jnp.clip(pseudo, 0.0, 1.0)
    xs = x[src]
    E = src.shape[0]
    msg = 0.0
    for combo in range(2 ** DIM):
        bb = jnp.ones((E,), dtype=x.dtype)
        kidx = 0
        for d in range(DIM):
            bit = (combo >> d) & 1
            bb = bb * (frac[:, d] if bit else (1.0 - frac[:, d]))
            kidx += bit * (K_SIZE ** d)
        msg = msg + bb[:, None] * (xs @ W[kidx])
    dstm = jnp.where(valid, dst, N)
    deg = jax.ops.segment_sum(valid.astype(x.dtype), dstm, num_segments=N + 1)[:N]
    agg = jax.ops.segment_sum(jnp.where(valid[:, None], msg, 0.0), dstm, num_segments=N + 1)[:N]
    return agg / jnp.maximum(deg, 1.0)[:, None]


def kernel(x, pos, edge_index, batch, W1, W2, W3, W4, W5, W6, W7,
           g1, g2, g3, g4, g5, g6, g7, b1, b2, b3, b4, b5, b6, b7, Wfc):
    Ws = [W1, W2, W3, W4, W5, W6, W7]
    gs = [g1, g2, g3, g4, g5, g6, g7]
    bs = [b1, b2, b3, b4, b5, b6, b7]
    batch = batch.astype(jnp.int32)
    ei = edge_index.astype(jnp.int32)
    B = GRID_B
    N1 = B * 4096
    c = jnp.clip(jnp.floor(pos * 16.0).astype(jnp.int32), 0, 15)
    inv1 = batch * 4096 + c[:, 0] * 256 + c[:, 1] * 16 + c[:, 2]
    cnt1 = jax.ops.segment_sum(jnp.ones(inv1.shape, jnp.float32), inv1, num_segments=N1)
    occ1 = cnt1 > 0
    E = ei.shape[1]
    s1, d1, v1 = _coarsen(inv1[ei[0]], inv1[ei[1]], jnp.ones((E,), dtype=bool), N1)
    p1 = jax.ops.segment_sum(pos, inv1, num_segments=N1) / jnp.maximum(cnt1, 1.0)[:, None]
    batch1 = jnp.arange(N1, dtype=jnp.int32) // 4096
    N2 = B * 64
    c2 = jnp.clip(jnp.floor(p1[:, :2] * 8.0).astype(jnp.int32), 0, 7)
    inv2 = jnp.where(occ1, batch1 * 64 + c2[:, 0] * 8 + c2[:, 1], N2)
    cnt2 = jax.ops.segment_sum(occ1.astype(jnp.float32), inv2, num_segments=N2 + 1)[:N2]
    occ2 = cnt2 > 0
    s2, d2, v2 = _coarsen(inv2[s1], inv2[jnp.minimum(d1, N1 - 1)], v1, N2)
    p2 = jax.ops.segment_sum(p1, inv2, num_segments=N2 + 1)[:N2] / jnp.maximum(cnt2, 1.0)[:, None]
    batch2 = jnp.arange(N2, dtype=jnp.int32) // 64
    c3 = jnp.clip(jnp.floor(p2[:, :2] * 2.0).astype(jnp.int32), 0, 1)
    cl3 = jnp.where(occ2, batch2 * 4 + c3[:, 0] * 2 + c3[:, 1], B * 4)

    m1 = occ1.astype(x.dtype)
    n1 = jnp.sum(m1)
    h = jax.ops.segment_sum(x, inv1, num_segments=N1) / jnp.maximum(cnt1, 1.0)[:, None]
    ea1 = _cart_norm(p1, s1, d1, v1)
    h = _bn_elu(_sconv(h, s1, d1, v1, ea1, Ws[0], N1), m1, n1, gs[0], bs[0])
    h = _bn_elu(_sconv(h, s1, d1, v1, ea1, Ws[1], N1), m1, n1, gs[1], bs[1])
    sc = h
    h = _bn_elu(_sconv(h, s1, d1, v1, ea1, Ws[2], N1), m1, n1, gs[2], bs[2])
    h = _bn_elu(_sconv(h, s1, d1, v1, ea1, Ws[3], N1), m1, n1, gs[3], bs[3], res=sc)
    h = _bn_elu(_sconv(h, s1, d1, v1, ea1, Ws[4], N1), m1, n1, gs[4], bs[4])
    h = jax.ops.segment_max(h, inv2, num_segments=N2 + 1)[:N2]
    h = jnp.where(occ2[:, None], h, 0.0)
    m2 = occ2.astype(x.dtype)
    n2 = jnp.sum(m2)
    ea2 = _cart_norm(p2, s2, d2, v2)
    sc = h
    h = _bn_elu(_sconv(h, s2, d2, v2, ea2, Ws[5], N2), m2, n2, gs[5], bs[5])
    h = _bn_elu(_sconv(h, s2, d2, v2, ea2, Ws[6], N2), m2, n2, gs[6], bs[6], res=sc)
    pooled = jax.ops.segment_max(h, cl3, num_segments=B * 4 + 1)[:B * 4]
    pooled = jnp.where(jnp.isfinite(pooled), pooled, 0.0)
    return pooled.reshape(B, 4 * CHANNELS[7]) @ Wfc.T


# speedup vs baseline: 1.6008x; 1.6008x over previous
import functools
import jax
import jax.numpy as jnp
from jax import lax
from jax.experimental import pallas as pl
from jax.experimental.pallas import tpu as pltpu, tpu_sc as plsc

K_SIZE = 2
DIM = 3
CHANNELS = [1, 16, 32, 32, 32, 64, 64, 64]
NUM_CLASSES = 10
GRID_B = 4

NC, NS, L = 2, 16, 16
NW = NC * NS
EG = 128
NSLOT = 4


@functools.lru_cache(maxsize=None)
def _make_edge_agg(E_pad, N, cin):
    chunk = E_pad // NW
    assert chunk % EG == 0 and chunk * NW == E_pad
    assert cin % L == 0
    ACC = 8 * cin
    nvec = ACC // L
    mesh = plsc.VectorSubcoreMesh(core_axis_name="c", subcore_axis_name="s",
                                  num_cores=NC, num_subcores=NS)

    def body(xh, srcs_h, dsts_h, b8_h, prevd_h, nblk_h, acc_h,
             idx_v, dst_v, b8_v, rows_v, accs_v, prevd_v, nblk_v, gsem):
        wid = lax.axis_index("s") * NC + lax.axis_index("c")
        start = wid * chunk
        end = start + chunk
        pltpu.sync_copy(prevd_h, prevd_v)
        pltpu.sync_copy(nblk_h, nblk_v)
        prev = prevd_v[wid, :][0]
        nblk = nblk_v[wid, :][0]
        zv = jnp.zeros((L,), jnp.float32)
        for j in range(nvec):
            accs_v[0, j * L:(j + 1) * L] = zv

        def flush(gate, cur_dst):
            @pl.when(gate)
            def _():
                pltpu.sync_copy(accs_v, acc_h.at[pl.ds(cur_dst, 1)])
                for j in range(nvec):
                    accs_v[0, j * L:(j + 1) * L] = zv

        def edge_body(e, st):
            cur_dst, owned, fin = st
            d = dst_v[pl.ds(e, L)][0]
            is_new = d != cur_dst
            do_flush = is_new & (owned == 1) & (fin == 0)
            stop_now = is_new & (edge_body.off + e >= end) & (fin == 0)
            flush(do_flush, cur_dst)
            nowned = jnp.where(is_new & (fin == 0), 1, owned)
            nfin = jnp.where(stop_now, 1, fin)
            live = (nfin == 0) & (nowned == 1)
            bvec = b8_v[e, :]

            @pl.when(live)
            def _():
                for k in range(8):
                    bk = bvec[k]
                    for jl in range(cin // L):
                        o = k * cin + jl * L
                        accs_v[0, o:o + L] = (
                            accs_v[0, o:o + L]
                            + bk * rows_v[e, jl * L:(jl + 1) * L])

            ncur = jnp.where(is_new, d, cur_dst)
            return ncur, nowned, nfin

        def block_body(boff, st):
            off = start + boff * EG
            pltpu.sync_copy(srcs_h.at[pl.ds(off, EG)], idx_v)
            pltpu.sync_copy(dsts_h.at[pl.ds(off, EG)], dst_v.at[pl.ds(0, EG)])
            pltpu.sync_copy(b8_h.at[pl.ds(off, EG)], b8_v)
            pltpu.async_copy(xh.at[idx_v], rows_v, gsem).wait()
            edge_body.off = off
            return lax.fori_loop(0, EG, edge_body, st)

        st = lax.fori_loop(
            0, nblk, block_body,
            (prev, jnp.int32(0), jnp.int32(0)))
        cur_dst, owned, fin = st
        flush((owned == 1) & (fin == 0), cur_dst)

    kern = pl.kernel(
        body,
        out_type=jax.ShapeDtypeStruct((N, ACC), jnp.float32),
        mesh=mesh,
        scratch_types=(
            pltpu.MemorySpace.VMEM((EG,), jnp.int32),
            pltpu.MemorySpace.VMEM((EG + L,), jnp.int32),
            pltpu.MemorySpace.VMEM((EG, L), jnp.float32),
            pltpu.MemorySpace.VMEM((EG, cin), jnp.float32),
            pltpu.MemorySpace.VMEM((1, ACC), jnp.float32),
            pltpu.MemorySpace.VMEM((NW, L), jnp.int32),
            pltpu.MemorySpace.VMEM((NW, L), jnp.int32),
            pltpu.SemaphoreType.DMA,
        ),
        compiler_params=pltpu.CompilerParams(use_tc_tiling_on_sc=False),
    )
    return kern


def _edge_agg(xh, srcs, dsts, b16, prevd, nblk, N):
    E_pad = srcs.shape[0]
    cin = xh.shape[1]
    kern = _make_edge_agg(E_pad, N, cin)
    return kern(xh, srcs, dsts, b16, prevd, nblk)


def _mm_elu_body(acc_ref, w_ref, deg_ref, m_ref, e_ref, ps_ref):
    i = pl.program_id(0)
    a = jnp.dot(acc_ref[...], w_ref[...], preferred_element_type=jnp.float32)
    dg = deg_ref[...]
    a = jnp.where(dg > 0, a / jnp.maximum(dg, 1.0), 0.0)
    e = jnp.where(a > 0, a, jnp.exp(jnp.minimum(a, 0.0)) - 1.0)
    e_ref[...] = e
    em = e * m_ref[...]

    @pl.when(i == 0)
    def _():
        ps_ref[...] = jnp.zeros(ps_ref.shape, ps_ref.dtype)

    ps_ref[0:1, :] += jnp.sum(em, axis=0, keepdims=True)
    ps_ref[1:2, :] += jnp.sum(em * e, axis=0, keepdims=True)


def _bn_apply_body(e_ref, ps_ref, n_ref, g_ref, b_ref, o_ref, *maybe_res):
    n = n_ref[0, 0]
    mu = ps_ref[0:1, :] / n
    var = ps_ref[1:2, :] / n - mu * mu
    out = (e_ref[...] - mu) * jax.lax.rsqrt(var + 1e-5) * g_ref[...] \
        + b_ref[...]
    if maybe_res:
        out = out + maybe_res[0][...]
    o_ref[...] = out


def _bn_apply_body_res(e_ref, ps_ref, n_ref, g_ref, b_ref, r_ref, o_ref):
    _bn_apply_body(e_ref, ps_ref, n_ref, g_ref, b_ref, o_ref, r_ref)


def _conv_finish(acc, Wstk, deg, mask, n, g, b, res=None):
    N, ACC = acc.shape
    cout = Wstk.shape[1]
    blk = min(N, 2048)
    grid = N // blk
    e, ps = pl.pallas_call(
        _mm_elu_body,
        grid=(grid,),
        in_specs=[
            pl.BlockSpec((blk, ACC), lambda i: (i, 0)),
            pl.BlockSpec((ACC, cout), lambda i: (0, 0)),
            pl.BlockSpec((blk, 1), lambda i: (i, 0)),
            pl.BlockSpec((blk, 1), lambda i: (i, 0)),
        ],
        out_specs=[
            pl.BlockSpec((blk, cout), lambda i: (i, 0)),
            pl.BlockSpec((8, cout), lambda i: (0, 0)),
        ],
        out_shape=[
            jax.ShapeDtypeStruct((N, cout), jnp.float32),
            jax.ShapeDtypeStruct((8, cout), jnp.float32),
        ],
    )(acc, Wstk, deg[:, None], mask[:, None])
    n2 = jnp.reshape(n, (1, 1))
    g2 = g[None, :]
    b2 = b[None, :]
    out_shape = jax.ShapeDtypeStruct((N, cout), jnp.float32)
    bspec = [
        pl.BlockSpec((blk, cout), lambda i: (i, 0)),
        pl.BlockSpec((8, cout), lambda i: (0, 0)),
        pl.BlockSpec((1, 1), lambda i: (0, 0)),
        pl.BlockSpec((1, cout), lambda i: (0, 0)),
        pl.BlockSpec((1, cout), lambda i: (0, 0)),
    ]
    ospec = pl.BlockSpec((blk, cout), lambda i: (i, 0))
    if res is None:
        return pl.pallas_call(
            _bn_apply_body, grid=(grid,), in_specs=bspec, out_specs=ospec,
            out_shape=out_shape)(e, ps, n2, g2, b2)
    return pl.pallas_call(
        _bn_apply_body_res, grid=(grid,),
        in_specs=bspec + [pl.BlockSpec((blk, cout), lambda i: (i, 0))],
        out_specs=ospec, out_shape=out_shape)(e, ps, n2, g2, b2, res)


def _spline_w8(ea, valid):
    frac = jnp.clip(ea, 0.0, 1.0)
    cols = []
    for combo in range(8):
        bb = valid.astype(jnp.float32)
        for d in range(DIM):
            bit = (combo >> d) & 1
            bb = bb * (frac[:, d] if bit else (1.0 - frac[:, d]))
        cols.append(bb)
    b8 = jnp.stack(cols, axis=1)
    return jnp.pad(b8, ((0, 0), (0, 8)))


def _prevd(dsts, chunk):
    pd = dsts[chunk - 1::chunk][:NW - 1]
    return jnp.tile(
        jnp.concatenate([jnp.full((1,), -1, jnp.int32), pd])[:, None],
        (1, L))


def _nblocks(dsts, chunk):
    E_pad = dsts.shape[0]
    ar = jnp.arange(E_pad, dtype=jnp.int32)
    is_start = jnp.concatenate(
        [jnp.ones((1,), bool), dsts[1:] != dsts[:-1]])
    ridx = jnp.where(is_start, ar, E_pad)
    sm = lax.cummin(ridx[::-1])[::-1]
    ends = (jnp.arange(NW, dtype=jnp.int32) + 1) * chunk
    stop = jnp.concatenate([sm, jnp.full((1,), E_pad, jnp.int32)])[ends]
    starts = jnp.arange(NW, dtype=jnp.int32) * chunk
    nblk = (stop - starts + EG - 1) // EG
    return jnp.tile(nblk[:, None], (1, L))


def _stack_w(W):
    kk, cin, cout = W.shape
    if cin < L:
        W = jnp.pad(W, ((0, 0), (0, L - cin), (0, 0)))
        cin = L
    return W.reshape(kk * cin, cout)


def kernel(x, pos, edge_index, batch, W1, W2, W3, W4, W5, W6, W7,
           g1, g2, g3, g4, g5, g6, g7, b1, b2, b3, b4, b5, b6, b7, Wfc):
    Ws = [W1, W2, W3, W4, W5, W6, W7]
    gs = [g1, g2, g3, g4, g5, g6, g7]
    bs = [b1, b2, b3, b4, b5, b6, b7]
    batch = batch.astype(jnp.int32)
    ei = edge_index.astype(jnp.int32)
    B = GRID_B
    N1 = B * 4096
    E = ei.shape[1]

    c = jnp.clip(jnp.floor(pos * 16.0).astype(jnp.int32), 0, 15)
    inv1 = batch * 4096 + c[:, 0] * 256 + c[:, 1] * 16 + c[:, 2]
    cnt1 = jax.ops.segment_sum(jnp.ones(inv1.shape, jnp.float32), inv1,
                               num_segments=N1)
    occ1 = cnt1 > 0
    p1 = jax.ops.segment_sum(pos, inv1, num_segments=N1) / \
        jnp.maximum(cnt1, 1.0)[:, None]
    h = jax.ops.segment_sum(x, inv1, num_segments=N1) / \
        jnp.maximum(cnt1, 1.0)[:, None]
    h = jnp.pad(h, ((0, 0), (0, L - h.shape[1])))

    sn = inv1[ei[0]]
    dn = inv1[ei[1]]
    sentinel = N1 * N1
    key = jnp.where(sn != dn, dn * N1 + sn, sentinel)
    ks = jnp.sort(key)
    first = jnp.concatenate([jnp.ones((1,), bool), ks[1:] != ks[:-1]])
    valid1 = first & (ks != sentinel)
    dq = ks // N1
    dsts1 = jnp.minimum(dq, N1 - 1).astype(jnp.int32)
    srcs1 = jnp.where(valid1, ks - dq * N1, 0).astype(jnp.int32)
    deg1 = jax.ops.segment_sum(valid1.astype(jnp.float32), dsts1,
                               num_segments=N1)
    cart1 = jnp.where(valid1[:, None], p1[dsts1] - p1[srcs1], 0.0)
    mx1 = jnp.max(jnp.abs(cart1))
    ea1 = cart1 / (2.0 * mx1) + 0.5
    b16_1 = _spline_w8(ea1, valid1)
    E_pad = ((E + NW * EG - 1) // (NW * EG)) * (NW * EG)
    pad = E_pad - E
    valid1p = jnp.pad(valid1, (0, pad))
    srcs1 = jnp.pad(srcs1, (0, pad))
    dsts1 = jnp.pad(dsts1, (0, pad), constant_values=N1 - 1)
    b16_1 = jnp.pad(b16_1, ((0, pad), (0, 0)))
    prevd1 = _prevd(dsts1, E_pad // NW)

    m1 = occ1.astype(jnp.float32)
    n1 = jnp.sum(m1)

    def conv(hin, i, lvl, res=None):
        srcs, dsts, b16, prevd, nblk, deg, m, n, N = lvl
        acc = _edge_agg(hin, srcs, dsts, b16, prevd, nblk, N)
        return _conv_finish(acc, _stack_w(Ws[i]), deg, m, n, gs[i], bs[i],
                            res=res)

    nblk1 = _nblocks(dsts1, E_pad // NW)
    lvl1 = (srcs1, dsts1, b16_1, prevd1, nblk1, deg1, m1, n1, N1)
    h = conv(h, 0, lvl1)
    h = conv(h, 1, lvl1)
    sc = h
    h = conv(h, 2, lvl1)
    h = conv(h, 3, lvl1, res=sc)
    h = conv(h, 4, lvl1)

    batch1 = jnp.arange(N1, dtype=jnp.int32) // 4096
    N2 = B * 64
    c2 = jnp.clip(jnp.floor(p1[:, :2] * 8.0).astype(jnp.int32), 0, 7)
    inv2 = jnp.where(occ1, batch1 * 64 + c2[:, 0] * 8 + c2[:, 1], N2)
    cnt2 = jax.ops.segment_sum(occ1.astype(jnp.float32), inv2,
                               num_segments=N2 + 1)[:N2]
    occ2 = cnt2 > 0
    p2 = jax.ops.segment_sum(p1, inv2, num_segments=N2 + 1)[:N2] / \
        jnp.maximum(cnt2, 1.0)[:, None]

    s2n = inv2[srcs1]
    d2n = inv2[dsts1]
    key2 = jnp.where(valid1p & (s2n != d2n), d2n * N2 + s2n, N2 * N2)
    cnt2e = jax.ops.segment_sum(jnp.ones(key2.shape, jnp.float32), key2,
                                num_segments=N2 * N2 + 1)[:N2 * N2]
    valid2 = cnt2e > 0
    E2 = N2 * N2
    ar2 = jnp.arange(E2, dtype=jnp.int32)
    srcs2 = ar2 % N2
    dsts2 = ar2 // N2
    deg2 = jnp.sum(valid2.reshape(N2, N2).astype(jnp.float32), axis=1)
    cart2 = jnp.where(valid2[:, None], p2[dsts2] - p2[srcs2], 0.0)
    mx2 = jnp.max(jnp.abs(cart2))
    ea2 = cart2 / (2.0 * mx2) + 0.5
    b16_2 = _spline_w8(ea2, valid2)
    prevd2 = _prevd(dsts2, E2 // NW)

    h = jax.ops.segment_max(h, inv2, num_segments=N2 + 1)[:N2]
    h = jnp.where(occ2[:, None], h, 0.0)
    m2 = occ2.astype(jnp.float32)
    n2 = jnp.sum(m2)

    nblk2 = _nblocks(dsts2, E2 // NW)
    lvl2 = (srcs2, dsts2, b16_2, prevd2, nblk2, deg2, m2, n2, N2)
    sc = h
    h = conv(h, 5, lvl2)
    h = conv(h, 6, lvl2, res=sc)

    batch2 = jnp.arange(N2, dtype=jnp.int32) // 64
    c3 = jnp.clip(jnp.floor(p2[:, :2] * 2.0).astype(jnp.int32), 0, 1)
    cl3 = jnp.where(occ2, batch2 * 4 + c3[:, 0] * 2 + c3[:, 1], B * 4)
    pooled = jax.ops.segment_max(h, cl3, num_segments=B * 4 + 1)[:B * 4]
    pooled = jnp.where(jnp.isfinite(pooled), pooled, 0.0)
    return pooled.reshape(B, 4 * CHANNELS[7]) @ Wfc.T


# speedup vs baseline: 1.8851x; 1.1776x over previous
import functools
import jax
import jax.numpy as jnp
from jax import lax
from jax.experimental import pallas as pl
from jax.experimental.pallas import tpu as pltpu, tpu_sc as plsc

K_SIZE = 2
DIM = 3
CHANNELS = [1, 16, 32, 32, 32, 64, 64, 64]
NUM_CLASSES = 10
GRID_B = 4

NC, NS, L = 2, 16, 16
NW = NC * NS
EG = 512
GSUB = 128


@functools.lru_cache(maxsize=None)
def _make_edge_agg(E_pad, N, cin):
    chunk = E_pad // NW
    assert chunk % EG == 0 and chunk * NW == E_pad
    assert cin % L == 0
    ACC = 8 * cin
    nvec = ACC // L
    mesh = plsc.VectorSubcoreMesh(core_axis_name="c", subcore_axis_name="s",
                                  num_cores=NC, num_subcores=NS)

    def body(xh, srcs_h, dsts_h, b8_h, prevd_h, nblk_h, acc_h,
             idx_v, dst_v, b8_v, rows_v, accs_v, prevd_v, nblk_v, gsem):
        wid = lax.axis_index("s") * NC + lax.axis_index("c")
        start = wid * chunk
        end = start + chunk
        pltpu.sync_copy(prevd_h, prevd_v)
        pltpu.sync_copy(nblk_h, nblk_v)
        prev = prevd_v[wid, :][0]
        nblk = nblk_v[wid, :][0]
        zv = jnp.zeros((L,), jnp.float32)
        for j in range(nvec):
            accs_v[0, j * L:(j + 1) * L] = zv

        def flush(gate, cur_dst):
            @pl.when(gate)
            def _():
                pltpu.sync_copy(accs_v, acc_h.at[pl.ds(cur_dst, 1)])
                for j in range(nvec):
                    accs_v[0, j * L:(j + 1) * L] = zv

        def edge_body(e, st):
            cur_dst, owned, fin = st
            d = dst_v[pl.ds(e, L)][0]
            is_new = d != cur_dst
            do_flush = is_new & (owned == 1) & (fin == 0)
            stop_now = is_new & (edge_body.off + e >= end) & (fin == 0)
            flush(do_flush, cur_dst)
            nowned = jnp.where(is_new & (fin == 0), 1, owned)
            nfin = jnp.where(stop_now, 1, fin)
            live = (nfin == 0) & (nowned == 1)
            bvec = b8_v[e, :]
            rowv = [rows_v[e // GSUB, e % GSUB, jl * L:(jl + 1) * L]
                    for jl in range(cin // L)]

            @pl.when(live)
            def _():
                for k in range(8):
                    bk = bvec[k]
                    for jl in range(cin // L):
                        o = k * cin + jl * L
                        plsc.addupdate(accs_v.at[0, pl.ds(o, L)],
                                       bk * rowv[jl])

            ncur = jnp.where(is_new, d, cur_dst)
            return ncur, nowned, nfin

        def block_body(boff, st):
            off = start + boff * EG
            for g in range(EG // GSUB):
                pltpu.sync_copy(srcs_h.at[pl.ds(off + g * GSUB, GSUB)],
                                idx_v.at[g])
            pltpu.sync_copy(dsts_h.at[pl.ds(off, EG)], dst_v.at[pl.ds(0, EG)])
            pltpu.sync_copy(b8_h.at[pl.ds(off, EG)], b8_v)
            cps = [pltpu.async_copy(
                xh.at[idx_v.at[g]], rows_v.at[g], gsem)
                for g in range(EG // GSUB)]
            for c in cps:
                c.wait()
            edge_body.off = off
            return lax.fori_loop(0, EG, edge_body, st)

        st = lax.fori_loop(
            0, nblk, block_body,
            (prev, jnp.int32(0), jnp.int32(0)))
        cur_dst, owned, fin = st
        flush((owned == 1) & (fin == 0), cur_dst)

    kern = pl.kernel(
        body,
        out_type=jax.ShapeDtypeStruct((N, ACC), jnp.float32),
        mesh=mesh,
        scratch_types=(
            pltpu.MemorySpace.VMEM((EG // GSUB, GSUB), jnp.int32),
            pltpu.MemorySpace.VMEM((EG + L,), jnp.int32),
            pltpu.MemorySpace.VMEM((EG, L), jnp.float32),
            pltpu.MemorySpace.VMEM((EG // GSUB, GSUB, cin), jnp.float32),
            pltpu.MemorySpace.VMEM((1, ACC), jnp.float32),
            pltpu.MemorySpace.VMEM((NW, L), jnp.int32),
            pltpu.MemorySpace.VMEM((NW, L), jnp.int32),
            pltpu.SemaphoreType.DMA,
        ),
        compiler_params=pltpu.CompilerParams(use_tc_tiling_on_sc=False),
    )
    return kern


def _edge_agg(xh, srcs, dsts, b16, prevd, nblk, N):
    E_pad = srcs.shape[0]
    cin = xh.shape[1]
    kern = _make_edge_agg(E_pad, N, cin)
    return kern(xh, srcs, dsts, b16, prevd, nblk)


def _mm_elu_body(acc_ref, w_ref, deg_ref, m_ref, e_ref, ps_ref):
    i = pl.program_id(0)
    a = jnp.dot(acc_ref[...], w_ref[...], preferred_element_type=jnp.float32)
    dg = deg_ref[...]
    a = jnp.where(dg > 0, a / jnp.maximum(dg, 1.0), 0.0)
    e = jnp.where(a > 0, a, jnp.exp(jnp.minimum(a, 0.0)) - 1.0)
    e_ref[...] = e
    em = e * m_ref[...]

    @pl.when(i == 0)
    def _():
        ps_ref[...] = jnp.zeros(ps_ref.shape, ps_ref.dtype)

    ps_ref[0:1, :] += jnp.sum(em, axis=0, keepdims=True)
    ps_ref[1:2, :] += jnp.sum(em * e, axis=0, keepdims=True)


def _bn_apply_body(e_ref, ps_ref, n_ref, g_ref, b_ref, o_ref, *maybe_res):
    n = n_ref[0, 0]
    mu = ps_ref[0:1, :] / n
    var = ps_ref[1:2, :] / n - mu * mu
    out = (e_ref[...] - mu) * jax.lax.rsqrt(var + 1e-5) * g_ref[...] \
        + b_ref[...]
    if maybe_res:
        out = out + maybe_res[0][...]
    o_ref[...] = out


def _bn_apply_body_res(e_ref, ps_ref, n_ref, g_ref, b_ref, r_ref, o_ref):
    _bn_apply_body(e_ref, ps_ref, n_ref, g_ref, b_ref, o_ref, r_ref)


def _conv_finish(acc, Wstk, deg, mask, n, g, b, res=None):
    N, ACC = acc.shape
    cout = Wstk.shape[1]
    blk = min(N, 2048)
    grid = N // blk
    e, ps = pl.pallas_call(
        _mm_elu_body,
        grid=(grid,),
        in_specs=[
            pl.BlockSpec((blk, ACC), lambda i: (i, 0)),
            pl.BlockSpec((ACC, cout), lambda i: (0, 0)),
            pl.BlockSpec((blk, 1), lambda i: (i, 0)),
            pl.BlockSpec((blk, 1), lambda i: (i, 0)),
        ],
        out_specs=[
            pl.BlockSpec((blk, cout), lambda i: (i, 0)),
            pl.BlockSpec((8, cout), lambda i: (0, 0)),
        ],
        out_shape=[
            jax.ShapeDtypeStruct((N, cout), jnp.float32),
            jax.ShapeDtypeStruct((8, cout), jnp.float32),
        ],
    )(acc, Wstk, deg[:, None], mask[:, None])
    n2 = jnp.reshape(n, (1, 1))
    g2 = g[None, :]
    b2 = b[None, :]
    out_shape = jax.ShapeDtypeStruct((N, cout), jnp.float32)
    bspec = [
        pl.BlockSpec((blk, cout), lambda i: (i, 0)),
        pl.BlockSpec((8, cout), lambda i: (0, 0)),
        pl.BlockSpec((1, 1), lambda i: (0, 0)),
        pl.BlockSpec((1, cout), lambda i: (0, 0)),
        pl.BlockSpec((1, cout), lambda i: (0, 0)),
    ]
    ospec = pl.BlockSpec((blk, cout), lambda i: (i, 0))
    if res is None:
        return pl.pallas_call(
            _bn_apply_body, grid=(grid,), in_specs=bspec, out_specs=ospec,
            out_shape=out_shape)(e, ps, n2, g2, b2)
    return pl.pallas_call(
        _bn_apply_body_res, grid=(grid,),
        in_specs=bspec + [pl.BlockSpec((blk, cout), lambda i: (i, 0))],
        out_specs=ospec, out_shape=out_shape)(e, ps, n2, g2, b2, res)


def _spline_w8(ea, valid):
    frac = jnp.clip(ea, 0.0, 1.0)
    cols = []
    for combo in range(8):
        bb = valid.astype(jnp.float32)
        for d in range(DIM):
            bit = (combo >> d) & 1
            bb = bb * (frac[:, d] if bit else (1.0 - frac[:, d]))
        cols.append(bb)
    b8 = jnp.stack(cols, axis=1)
    return jnp.pad(b8, ((0, 0), (0, 8)))


def _prevd(dsts, chunk):
    pd = dsts[chunk - 1::chunk][:NW - 1]
    return jnp.tile(
        jnp.concatenate([jnp.full((1,), -1, jnp.int32), pd])[:, None],
        (1, L))


def _nblocks(dsts, chunk):
    E_pad = dsts.shape[0]
    ar = jnp.arange(E_pad, dtype=jnp.int32)
    is_start = jnp.concatenate(
        [jnp.ones((1,), bool), dsts[1:] != dsts[:-1]])
    ridx = jnp.where(is_start, ar, E_pad)
    sm = lax.cummin(ridx[::-1])[::-1]
    ends = (jnp.arange(NW, dtype=jnp.int32) + 1) * chunk
    stop = jnp.concatenate([sm, jnp.full((1,), E_pad, jnp.int32)])[ends]
    starts = jnp.arange(NW, dtype=jnp.int32) * chunk
    nblk = (stop - starts + EG - 1) // EG
    return jnp.tile(nblk[:, None], (1, L))


def _stack_w(W):
    kk, cin, cout = W.shape
    if cin < L:
        W = jnp.pad(W, ((0, 0), (0, L - cin), (0, 0)))
        cin = L
    return W.reshape(kk * cin, cout)


def kernel(x, pos, edge_index, batch, W1, W2, W3, W4, W5, W6, W7,
           g1, g2, g3, g4, g5, g6, g7, b1, b2, b3, b4, b5, b6, b7, Wfc):
    Ws = [W1, W2, W3, W4, W5, W6, W7]
    gs = [g1, g2, g3, g4, g5, g6, g7]
    bs = [b1, b2, b3, b4, b5, b6, b7]
    batch = batch.astype(jnp.int32)
    ei = edge_index.astype(jnp.int32)
    B = GRID_B
    N1 = B * 4096
    E = ei.shape[1]

    c = jnp.clip(jnp.floor(pos * 16.0).astype(jnp.int32), 0, 15)
    inv1 = batch * 4096 + c[:, 0] * 256 + c[:, 1] * 16 + c[:, 2]
    cnt1 = jax.ops.segment_sum(jnp.ones(inv1.shape, jnp.float32), inv1,
                               num_segments=N1)
    occ1 = cnt1 > 0
    p1 = jax.ops.segment_sum(pos, inv1, num_segments=N1) / \
        jnp.maximum(cnt1, 1.0)[:, None]
    h = jax.ops.segment_sum(x, inv1, num_segments=N1) / \
        jnp.maximum(cnt1, 1.0)[:, None]
    h = jnp.pad(h, ((0, 0), (0, L - h.shape[1])))

    sn = inv1[ei[0]]
    dn = inv1[ei[1]]
    sentinel = N1 * N1
    key = jnp.where(sn != dn, dn * N1 + sn, sentinel)
    ks = jnp.sort(key)
    first = jnp.concatenate([jnp.ones((1,), bool), ks[1:] != ks[:-1]])
    valid1 = first & (ks != sentinel)
    dq = ks // N1
    dsts1 = jnp.minimum(dq, N1 - 1).astype(jnp.int32)
    srcs1 = jnp.where(valid1, ks - dq * N1, 0).astype(jnp.int32)
    deg1 = jax.ops.segment_sum(valid1.astype(jnp.float32), dsts1,
                               num_segments=N1)
    cart1 = jnp.where(valid1[:, None], p1[dsts1] - p1[srcs1], 0.0)
    mx1 = jnp.max(jnp.abs(cart1))
    ea1 = cart1 / (2.0 * mx1) + 0.5
    b16_1 = _spline_w8(ea1, valid1)
    E_pad = ((E + NW * EG - 1) // (NW * EG)) * (NW * EG)
    pad = E_pad - E
    valid1p = jnp.pad(valid1, (0, pad))
    srcs1 = jnp.pad(srcs1, (0, pad))
    dsts1 = jnp.pad(dsts1, (0, pad), constant_values=N1 - 1)
    b16_1 = jnp.pad(b16_1, ((0, pad), (0, 0)))
    prevd1 = _prevd(dsts1, E_pad // NW)

    m1 = occ1.astype(jnp.float32)
    n1 = jnp.sum(m1)

    def conv(hin, i, lvl, res=None):
        srcs, dsts, b16, prevd, nblk, deg, m, n, N = lvl
        acc = _edge_agg(hin, srcs, dsts, b16, prevd, nblk, N)
        return _conv_finish(acc, _stack_w(Ws[i]), deg, m, n, gs[i], bs[i],
                            res=res)

    nblk1 = _nblocks(dsts1, E_pad // NW)
    lvl1 = (srcs1, dsts1, b16_1, prevd1, nblk1, deg1, m1, n1, N1)
    h = conv(h, 0, lvl1)
    h = conv(h, 1, lvl1)
    sc = h
    h = conv(h, 2, lvl1)
    h = conv(h, 3, lvl1, res=sc)
    h = conv(h, 4, lvl1)

    batch1 = jnp.arange(N1, dtype=jnp.int32) // 4096
    N2 = B * 64
    c2 = jnp.clip(jnp.floor(p1[:, :2] * 8.0).astype(jnp.int32), 0, 7)
    inv2 = jnp.where(occ1, batch1 * 64 + c2[:, 0] * 8 + c2[:, 1], N2)
    cnt2 = jax.ops.segment_sum(occ1.astype(jnp.float32), inv2,
                               num_segments=N2 + 1)[:N2]
    occ2 = cnt2 > 0
    p2 = jax.ops.segment_sum(p1, inv2, num_segments=N2 + 1)[:N2] / \
        jnp.maximum(cnt2, 1.0)[:, None]

    s2n = inv2[srcs1]
    d2n = inv2[dsts1]
    key2 = jnp.where(valid1p & (s2n != d2n), d2n * N2 + s2n, N2 * N2)
    cnt2e = jax.ops.segment_sum(jnp.ones(key2.shape, jnp.float32), key2,
                                num_segments=N2 * N2 + 1)[:N2 * N2]
    valid2 = cnt2e > 0
    E2 = N2 * N2
    ar2 = jnp.arange(E2, dtype=jnp.int32)
    srcs2 = ar2 % N2
    dsts2 = ar2 // N2
    deg2 = jnp.sum(valid2.reshape(N2, N2).astype(jnp.float32), axis=1)
    cart2 = jnp.where(valid2[:, None], p2[dsts2] - p2[srcs2], 0.0)
    mx2 = jnp.max(jnp.abs(cart2))
    ea2 = cart2 / (2.0 * mx2) + 0.5
    b16_2 = _spline_w8(ea2, valid2)
    prevd2 = _prevd(dsts2, E2 // NW)

    h = jax.ops.segment_max(h, inv2, num_segments=N2 + 1)[:N2]
    h = jnp.where(occ2[:, None], h, 0.0)
    m2 = occ2.astype(jnp.float32)
    n2 = jnp.sum(m2)

    nblk2 = _nblocks(dsts2, E2 // NW)
    lvl2 = (srcs2, dsts2, b16_2, prevd2, nblk2, deg2, m2, n2, N2)
    sc = h
    h = conv(h, 5, lvl2)
    h = conv(h, 6, lvl2, res=sc)

    batch2 = jnp.arange(N2, dtype=jnp.int32) // 64
    c3 = jnp.clip(jnp.floor(p2[:, :2] * 2.0).astype(jnp.int32), 0, 1)
    cl3 = jnp.where(occ2, batch2 * 4 + c3[:, 0] * 2 + c3[:, 1], B * 4)
    pooled = jax.ops.segment_max(h, cl3, num_segments=B * 4 + 1)[:B * 4]
    pooled = jnp.where(jnp.isfinite(pooled), pooled, 0.0)
    return pooled.reshape(B, 4 * CHANNELS[7]) @ Wfc.T
